# baseline (speedup 1.0000x reference)
import functools

import jax
import jax.numpy as jnp
from jax import lax
from jax.experimental import pallas as pl
from jax.experimental.pallas import tpu as pltpu
from jax.experimental.pallas import tpu_sc as plsc

N = 100000
L = 16
EMB = 64
TOKEN_VOCAB = 100000
NODE_VOCAB = 1000

NC = 2
NS = 16
NW = NC * NS
C = 32
SEG = C * L // 128
PW = 3136
K = PW // C


def _proj_token_body(a_ref, w_ref, o_ref):
    d = lax.dot_general(
        a_ref[...], w_ref[...], (((0,), (0,)), ((), ())),
        preferred_element_type=jnp.float32,
    )
    o_ref[:, 0:EMB] = d
    o_ref[:, EMB:2 * EMB] = d


def _proj_node_body(a_ref, w_ref, b_ref, o_ref):
    d = lax.dot_general(
        a_ref[...], w_ref[...], (((0,), (0,)), ((), ())),
        preferred_element_type=jnp.float32,
    ) + b_ref[...]
    o_ref[:, 0:EMB] = d
    o_ref[:, EMB:2 * EMB] = d


_TOK_BLK = 2000


_TV_PAD = 102400
_NV_PAD = 1024


def _project_tables(token_table, node_table, W, b):
    ttp = jnp.pad(token_table.T, ((0, 0), (0, _TV_PAD - TOKEN_VOCAB)))
    ntp = jnp.pad(node_table.T, ((0, 0), (0, _NV_PAD - NODE_VOCAB)))
    blk = _TV_PAD // 16
    pt2 = pl.pallas_call(
        _proj_token_body,
        grid=(16,),
        in_specs=[
            pl.BlockSpec((EMB, blk), lambda i: (0, i)),
            pl.BlockSpec((EMB, EMB), lambda i: (0, 0)),
        ],
        out_specs=pl.BlockSpec((blk, 2 * EMB), lambda i: (i, 0)),
        out_shape=jax.ShapeDtypeStruct((_TV_PAD, 2 * EMB), jnp.float32),
    )(ttp, W[:EMB])
    pn2 = pl.pallas_call(
        _proj_node_body,
        grid=(1,),
        in_specs=[
            pl.BlockSpec((EMB, _NV_PAD), lambda i: (0, 0)),
            pl.BlockSpec((EMB, EMB), lambda i: (0, 0)),
            pl.BlockSpec((1, EMB), lambda i: (0, 0)),
        ],
        out_specs=pl.BlockSpec((_NV_PAD, 2 * EMB), lambda i: (0, 0)),
        out_shape=jax.ShapeDtypeStruct((_NV_PAD, 2 * EMB), jnp.float32),
    )(ntp, W[EMB:], b.reshape(1, EMB))
    return pt2.reshape(2 * _TV_PAD, EMB), pn2.reshape(2 * _NV_PAD, EMB)


_mesh = plsc.VectorSubcoreMesh(core_axis_name="c", subcore_axis_name="s")


NH = N
PWH = PW
KH = PWH // C


def _make_sc_half(off):
    @functools.partial(
        pl.kernel,
        mesh=_mesh,
        out_type=jax.ShapeDtypeStruct((NH, EMB), jnp.float32),
        compiler_params=pltpu.CompilerParams(use_tc_tiling_on_sc=False),
        scratch_types=[
            pltpu.VMEM((2, SEG, 128), jnp.int32),
            pltpu.VMEM((2, C * L, EMB), jnp.float32),
            pltpu.VMEM((2, C), jnp.int32),
            pltpu.VMEM((2, C, EMB), jnp.float32),
            pltpu.VMEM((2, C, EMB), jnp.float32),
            pltpu.SemaphoreType.DMA,
            pltpu.SemaphoreType.DMA,
            pltpu.SemaphoreType.DMA,
            pltpu.SemaphoreType.DMA,
        ],
    )
    def _sc_half(tok1d, ntypes, pt, pn, out, idx_v, rows_v, nidx_v, nrows_v, out_v,
                 sem_a, sem_b, isem_a, isem_b):
        _sc_body(off, tok1d, ntypes, pt, pn, out, idx_v, rows_v, nidx_v, nrows_v,
                 out_v, (sem_a, sem_b), (isem_a, isem_b))

    return _sc_half


def _sc_body(off, tok1d, ntypes, pt, pn, out, idx_v, rows_v, nidx_v, nrows_v,
             out_v, sems, isems):
    wid = lax.axis_index("s") * NC + lax.axis_index("c")
    wbase = wid * PWH

    def chunk_base(k):
        return pl.multiple_of(jnp.minimum(wbase + k * C, NH - C), 16)

    def fire(k, buf):
        base = chunk_base(k) + off
        icopies = [pltpu.async_copy(ntypes.at[pl.ds(base, C)], nidx_v.at[buf], isems[buf])]
        for l in range(L):
            icopies.append(
                pltpu.async_copy(
                    tok1d.at[pl.ds(l * N + base, C)],
                    idx_v.at[buf].at[l * C // 128].at[pl.ds(l * C % 128, C)],
                    isems[buf],
                )
            )
        for cp in icopies:
            cp.wait()
        pltpu.async_copy(pn.at[nidx_v.at[buf]], nrows_v.at[buf], sems[buf])
        for j in range(SEG):
            pltpu.async_copy(
                pt.at[idx_v.at[buf].at[j]],
                rows_v.at[buf].at[pl.ds(j * 128, 128)],
                sems[buf],
            )

    def drain(buf):
        pltpu.make_async_copy(pn.at[nidx_v.at[buf]], nrows_v.at[buf], sems[buf]).wait()
        for j in range(SEG):
            pltpu.make_async_copy(
                pt.at[idx_v.at[buf].at[j]],
                rows_v.at[buf].at[pl.ds(j * 128, 128)],
                sems[buf],
            ).wait()

    def compute(k, buf):
        def node_body(n, carry2):
            acc = [nrows_v[buf, n, pl.ds(d * 16, 16)] for d in range(EMB // 16)]
            for l in range(L):
                for d in range(EMB // 16):
                    acc[d] = acc[d] + rows_v[buf, C * l + n, pl.ds(d * 16, 16)]
            for d in range(EMB // 16):
                out_v[buf, n, pl.ds(d * 16, 16)] = acc[d]
            return carry2

        lax.fori_loop(0, C, node_body, 0)
        pltpu.sync_copy(out_v.at[buf], out.at[pl.ds(chunk_base(k), C)])

    fire(0, 0)

    def pair_body(i, carry):
        k0 = 2 * i
        fire(k0 + 1, 1)
        drain(0)
        compute(k0, 0)

        @pl.when(i < KH // 2 - 1)
        def _():
            fire(k0 + 2, 0)

        drain(1)
        compute(k0 + 1, 1)
        return carry

    lax.fori_loop(0, KH // 2, pair_body, 0)


_sc_embed = _make_sc_half(0)


def kernel(tokens, node_types, token_table, node_table, W, b):
    pt, pn = _project_tables(token_table, node_table, W, b)
    tok1d = (tokens.astype(jnp.int32) * 2).T.reshape(N * L)
    return _sc_embed(tok1d, node_types.astype(jnp.int32) * 2, pt, pn)


# speedup vs baseline: 1.0063x; 1.0063x over previous
import functools

import jax
import jax.numpy as jnp
from jax import lax
from jax.experimental import pallas as pl
from jax.experimental.pallas import tpu as pltpu
from jax.experimental.pallas import tpu_sc as plsc

N = 100000
L = 16
EMB = 64
TOKEN_VOCAB = 100000
NODE_VOCAB = 1000

NC = 2
NS = 16
NW = NC * NS
C = 32
SEG = C * L // 128
PW = 3136
K = PW // C


def _proj_token_body(a_ref, w_ref, o_ref):
    d = lax.dot_general(
        a_ref[...], w_ref[...], (((0,), (0,)), ((), ())),
        preferred_element_type=jnp.float32,
    )
    o_ref[:, 0:EMB] = d
    o_ref[:, EMB:2 * EMB] = d


def _proj_node_body(a_ref, w_ref, b_ref, o_ref):
    d = lax.dot_general(
        a_ref[...], w_ref[...], (((0,), (0,)), ((), ())),
        preferred_element_type=jnp.float32,
    ) + b_ref[...]
    o_ref[:, 0:EMB] = d
    o_ref[:, EMB:2 * EMB] = d


_TOK_BLK = 2000


_TV_PAD = 102400
_NV_PAD = 1024


def _project_tables(token_table, node_table, W, b):
    ttp = jnp.pad(token_table.T, ((0, 0), (0, _TV_PAD - TOKEN_VOCAB)))
    ntp = jnp.pad(node_table.T, ((0, 0), (0, _NV_PAD - NODE_VOCAB)))
    blk = _TV_PAD // 16
    pt2 = pl.pallas_call(
        _proj_token_body,
        grid=(16,),
        in_specs=[
            pl.BlockSpec((EMB, blk), lambda i: (0, i)),
            pl.BlockSpec((EMB, EMB), lambda i: (0, 0)),
        ],
        out_specs=pl.BlockSpec((blk, 2 * EMB), lambda i: (i, 0)),
        out_shape=jax.ShapeDtypeStruct((_TV_PAD, 2 * EMB), jnp.float32),
    )(ttp, W[:EMB])
    pn2 = pl.pallas_call(
        _proj_node_body,
        grid=(1,),
        in_specs=[
            pl.BlockSpec((EMB, _NV_PAD), lambda i: (0, 0)),
            pl.BlockSpec((EMB, EMB), lambda i: (0, 0)),
            pl.BlockSpec((1, EMB), lambda i: (0, 0)),
        ],
        out_specs=pl.BlockSpec((_NV_PAD, 2 * EMB), lambda i: (0, 0)),
        out_shape=jax.ShapeDtypeStruct((_NV_PAD, 2 * EMB), jnp.float32),
    )(ntp, W[EMB:], b.reshape(1, EMB))
    return pt2.reshape(2 * _TV_PAD, EMB), pn2.reshape(2 * _NV_PAD, EMB)


_mesh = plsc.VectorSubcoreMesh(core_axis_name="c", subcore_axis_name="s")


NH = N
PWH = PW
KH = PWH // C


def _make_sc_half(off):
    @functools.partial(
        pl.kernel,
        mesh=_mesh,
        out_type=jax.ShapeDtypeStruct((NH, EMB), jnp.float32),
        compiler_params=pltpu.CompilerParams(use_tc_tiling_on_sc=False),
        scratch_types=[
            pltpu.VMEM((3, SEG, 128), jnp.int32),
            pltpu.VMEM((3, C * L, EMB), jnp.float32),
            pltpu.VMEM((3, C), jnp.int32),
            pltpu.VMEM((3, C, EMB), jnp.float32),
            pltpu.VMEM((3, C, EMB), jnp.float32),
            pltpu.SemaphoreType.DMA,
            pltpu.SemaphoreType.DMA,
            pltpu.SemaphoreType.DMA,
            pltpu.SemaphoreType.DMA,
            pltpu.SemaphoreType.DMA,
            pltpu.SemaphoreType.DMA,
        ],
    )
    def _sc_half(tok1d, ntypes, pt, pn, out, idx_v, rows_v, nidx_v, nrows_v, out_v,
                 s0, s1, s2, i0, i1, i2):
        _sc_body(off, tok1d, ntypes, pt, pn, out, idx_v, rows_v, nidx_v, nrows_v,
                 out_v, (s0, s1, s2), (i0, i1, i2))

    return _sc_half


def _sc_body(off, tok1d, ntypes, pt, pn, out, idx_v, rows_v, nidx_v, nrows_v,
             out_v, sems, isems):
    wid = lax.axis_index("s") * NC + lax.axis_index("c")
    wbase = wid * PWH

    def chunk_base(k):
        return pl.multiple_of(jnp.minimum(wbase + k * C, NH - C), 16)

    def fire(k, buf):
        base = chunk_base(k) + off
        icopies = [pltpu.async_copy(ntypes.at[pl.ds(base, C)], nidx_v.at[buf], isems[buf])]
        for l in range(L):
            icopies.append(
                pltpu.async_copy(
                    tok1d.at[pl.ds(l * N + base, C)],
                    idx_v.at[buf].at[l * C // 128].at[pl.ds(l * C % 128, C)],
                    isems[buf],
                )
            )
        for cp in icopies:
            cp.wait()
        pltpu.async_copy(pn.at[nidx_v.at[buf]], nrows_v.at[buf], sems[buf])
        for j in range(SEG):
            pltpu.async_copy(
                pt.at[idx_v.at[buf].at[j]],
                rows_v.at[buf].at[pl.ds(j * 128, 128)],
                sems[buf],
            )

    def drain(buf):
        pltpu.make_async_copy(pn.at[nidx_v.at[buf]], nrows_v.at[buf], sems[buf]).wait()
        for j in range(SEG):
            pltpu.make_async_copy(
                pt.at[idx_v.at[buf].at[j]],
                rows_v.at[buf].at[pl.ds(j * 128, 128)],
                sems[buf],
            ).wait()

    def compute(k, buf):
        def node_body(n, carry2):
            acc = [nrows_v[buf, n, pl.ds(d * 16, 16)] for d in range(EMB // 16)]
            for l in range(L):
                for d in range(EMB // 16):
                    acc[d] = acc[d] + rows_v[buf, C * l + n, pl.ds(d * 16, 16)]
            for d in range(EMB // 16):
                out_v[buf, n, pl.ds(d * 16, 16)] = acc[d]
            return carry2

        lax.fori_loop(0, C, node_body, 0)
        pltpu.sync_copy(out_v.at[buf], out.at[pl.ds(chunk_base(k), C)])

    fire(0, 0)
    fire(1, 1)
    fire(2, 2)

    def tri_body(i, carry):
        k0 = 3 * i
        for j in range(3):
            drain(j)
            compute(k0 + j, j)

            @pl.when(k0 + 3 + j < KH)
            def _(j=j):
                fire(k0 + 3 + j, j)

        return carry

    lax.fori_loop(0, KH // 3, tri_body, 0)
    for j in range(KH - 3 * (KH // 3)):
        drain(j)
        compute(KH - 2 + j, j)


_sc_embed = _make_sc_half(0)


def kernel(tokens, node_types, token_table, node_table, W, b):
    pt, pn = _project_tables(token_table, node_table, W, b)
    tok1d = (tokens.astype(jnp.int32) * 2).T.reshape(N * L)
    return _sc_embed(tok1d, node_types.astype(jnp.int32) * 2, pt, pn)


# speedup vs baseline: 1.0067x; 1.0004x over previous
import functools

import jax
import jax.numpy as jnp
from jax import lax
from jax.experimental import pallas as pl
from jax.experimental.pallas import tpu as pltpu
from jax.experimental.pallas import tpu_sc as plsc

N = 100000
L = 16
EMB = 64
TOKEN_VOCAB = 100000
NODE_VOCAB = 1000

NC = 2
NS = 16
NW = NC * NS
C = 32
SEG = C * L // 128
PW = 3136
K = PW // C


def _proj_token_body(a_ref, w_ref, o_ref):
    d = lax.dot_general(
        a_ref[...], w_ref[...], (((0,), (0,)), ((), ())),
        preferred_element_type=jnp.float32,
    )
    o_ref[:, 0:EMB] = d
    o_ref[:, EMB:2 * EMB] = d


def _proj_node_body(a_ref, w_ref, b_ref, o_ref):
    d = lax.dot_general(
        a_ref[...], w_ref[...], (((0,), (0,)), ((), ())),
        preferred_element_type=jnp.float32,
    ) + b_ref[...]
    o_ref[:, 0:EMB] = d
    o_ref[:, EMB:2 * EMB] = d


_TV_PAD = 102400
_NV_PAD = 1024


def _project_tables(token_table, node_table, W, b):
    ttp = jnp.pad(token_table.T, ((0, 0), (0, _TV_PAD - TOKEN_VOCAB)))
    ntp = jnp.pad(node_table.T, ((0, 0), (0, _NV_PAD - NODE_VOCAB)))
    blk = _TV_PAD // 16
    pt2 = pl.pallas_call(
        _proj_token_body,
        grid=(16,),
        in_specs=[
            pl.BlockSpec((EMB, blk), lambda i: (0, i)),
            pl.BlockSpec((EMB, EMB), lambda i: (0, 0)),
        ],
        out_specs=pl.BlockSpec((blk, 2 * EMB), lambda i: (i, 0)),
        out_shape=jax.ShapeDtypeStruct((_TV_PAD, 2 * EMB), jnp.float32),
    )(ttp, W[:EMB])
    pn2 = pl.pallas_call(
        _proj_node_body,
        grid=(1,),
        in_specs=[
            pl.BlockSpec((EMB, _NV_PAD), lambda i: (0, 0)),
            pl.BlockSpec((EMB, EMB), lambda i: (0, 0)),
            pl.BlockSpec((1, EMB), lambda i: (0, 0)),
        ],
        out_specs=pl.BlockSpec((_NV_PAD, 2 * EMB), lambda i: (0, 0)),
        out_shape=jax.ShapeDtypeStruct((_NV_PAD, 2 * EMB), jnp.float32),
    )(ntp, W[EMB:], b.reshape(1, EMB))
    return pt2.reshape(2 * _TV_PAD, EMB), pn2.reshape(2 * _NV_PAD, EMB)


_mesh = plsc.VectorSubcoreMesh(core_axis_name="c", subcore_axis_name="s")


NH = N
PWH = PW
KH = PWH // C


def _make_sc_kernel(off):
    @functools.partial(
        pl.kernel,
        mesh=_mesh,
        out_type=jax.ShapeDtypeStruct((NH, EMB), jnp.float32),
        compiler_params=pltpu.CompilerParams(use_tc_tiling_on_sc=False),
        scratch_types=[
            pltpu.VMEM((3, SEG, 128), jnp.int32),
            pltpu.VMEM((3, C * L, EMB), jnp.float32),
            pltpu.VMEM((3, C), jnp.int32),
            pltpu.VMEM((3, C, EMB), jnp.float32),
            pltpu.VMEM((3, C, EMB), jnp.float32),
            pltpu.SemaphoreType.DMA,
            pltpu.SemaphoreType.DMA,
            pltpu.SemaphoreType.DMA,
            pltpu.SemaphoreType.DMA,
            pltpu.SemaphoreType.DMA,
            pltpu.SemaphoreType.DMA,
        ],
    )
    def _sc_kernel(tok1d, ntypes, pt, pn, out, idx_v, rows_v, nidx_v, nrows_v, out_v,
                   s0, s1, s2, i0, i1, i2):
        _sc_body(off, tok1d, ntypes, pt, pn, out, idx_v, rows_v, nidx_v, nrows_v,
                 out_v, (s0, s1, s2), (i0, i1, i2))

    return _sc_kernel


def _sc_body(off, tok1d, ntypes, pt, pn, out, idx_v, rows_v, nidx_v, nrows_v,
             out_v, sems, isems):
    wid = lax.axis_index("s") * NC + lax.axis_index("c")
    wbase = wid * PWH

    def chunk_base(k):
        return pl.multiple_of(jnp.minimum(wbase + k * C, NH - C), 16)

    def fire(k, buf):
        base = chunk_base(k) + off
        icopies = [pltpu.async_copy(ntypes.at[pl.ds(base, C)], nidx_v.at[buf], isems[buf])]
        for l in range(L):
            icopies.append(
                pltpu.async_copy(
                    tok1d.at[pl.ds(l * N + base, C)],
                    idx_v.at[buf].at[l * C // 128].at[pl.ds(l * C % 128, C)],
                    isems[buf],
                )
            )
        for cp in icopies:
            cp.wait()
        pltpu.async_copy(pn.at[nidx_v.at[buf]], nrows_v.at[buf], sems[buf])
        for j in range(SEG):
            pltpu.async_copy(
                pt.at[idx_v.at[buf].at[j]],
                rows_v.at[buf].at[pl.ds(j * 128, 128)],
                sems[buf],
            )

    def drain(buf):
        pltpu.make_async_copy(pn.at[nidx_v.at[buf]], nrows_v.at[buf], sems[buf]).wait()
        for j in range(SEG):
            pltpu.make_async_copy(
                pt.at[idx_v.at[buf].at[j]],
                rows_v.at[buf].at[pl.ds(j * 128, 128)],
                sems[buf],
            ).wait()

    def compute(k, buf):
        def node_body(n, carry2):
            acc = [nrows_v[buf, n, pl.ds(d * 16, 16)] for d in range(EMB // 16)]
            for l in range(L):
                for d in range(EMB // 16):
                    acc[d] = acc[d] + rows_v[buf, C * l + n, pl.ds(d * 16, 16)]
            for d in range(EMB // 16):
                out_v[buf, n, pl.ds(d * 16, 16)] = acc[d]
            return carry2

        lax.fori_loop(0, C, node_body, 0)
        pltpu.sync_copy(out_v.at[buf], out.at[pl.ds(chunk_base(k), C)])

    fire(0, 0)
    fire(1, 1)
    fire(2, 2)

    def tri_body(i, carry):
        k0 = 3 * i
        for j in range(3):
            drain(j)
            compute(k0 + j, j)

            @pl.when(k0 + 3 + j < KH)
            def _(j=j):
                fire(k0 + 3 + j, j)

        return carry

    lax.fori_loop(0, KH // 3, tri_body, 0)
    for j in range(KH - 3 * (KH // 3)):
        drain(j)
        compute(KH - 2 + j, j)


_sc_embed = _make_sc_kernel(0)


def kernel(tokens, node_types, token_table, node_table, W, b):
    pt, pn = _project_tables(token_table, node_table, W, b)
    tok1d = (tokens.astype(jnp.int32) * 2).T.reshape(N * L)
    return _sc_embed(tok1d, node_types.astype(jnp.int32) * 2, pt, pn)
